# baseline (speedup 1.0000x reference)
import functools
import math

import jax
import jax.numpy as jnp
from jax import lax
from jax.experimental import pallas as pl
from jax.experimental.pallas import tpu as pltpu
from jax.experimental.pallas import tpu_sc as plsc

_PI = math.pi
_TWO_PI = 2.0 * math.pi


def _frac(x):
    r = lax.rem(x, jnp.float32(1.0))
    return jnp.where(r < 0.0, r + 1.0, r)


def _sqrt16(s):
    bits = plsc.bitcast(s, jnp.int32)
    seed = jnp.int32(0x5F3759DF) - lax.shift_right_logical(bits, 1)
    y = plsc.bitcast(seed, jnp.float32)
    for _ in range(3):
        y = y * (1.5 - 0.5 * s * y * y)
    return s * y


def _make_sc_call(n, d, rows_w, chunk):
    groups = chunk // 16
    n_chunks = rows_w // chunk
    mesh = plsc.VectorSubcoreMesh(core_axis_name="c", subcore_axis_name="s")

    @functools.partial(
        pl.kernel,
        out_type=jax.ShapeDtypeStruct((n, d), jnp.float32),
        mesh=mesh,
        scratch_types=[
            pltpu.VMEM((chunk, d), jnp.float32),
            pltpu.VMEM((chunk, d), jnp.float32),
            pltpu.VMEM((chunk, d), jnp.float32),
            pltpu.VMEM((chunk, d), jnp.float32),
            pltpu.SemaphoreType.DMA,
            pltpu.SemaphoreType.DMA,
            pltpu.SemaphoreType.DMA,
            pltpu.SemaphoreType.DMA,
            pltpu.VMEM((24,), jnp.float32),
            pltpu.VMEM((24,), jnp.float32),
        ],
        compiler_params=pltpu.CompilerParams(needs_layout_passes=False),
    )
    def sc_call(
        noise_hbm,
        means_hbm,
        stds_hbm,
        out_hbm,
        buf0,
        obuf0,
        buf1,
        obuf1,
        isem0,
        isem1,
        osem0,
        osem1,
        mean_b,
        std_b,
    ):
        bufs = (buf0, buf1)
        obufs = (obuf0, obuf1)
        isems = (isem0, isem1)
        osems = (osem0, osem1)
        nc = 2
        w = lax.axis_index("s") * nc + lax.axis_index("c")
        r0 = w * rows_w
        pltpu.sync_copy(means_hbm, mean_b.at[pl.ds(8, d)])
        pltpu.sync_copy(stds_hbm, std_b.at[pl.ds(8, d)])

        iota = lax.iota(jnp.int32, 16)
        cvec = [lax.rem(iota + g, jnp.int32(d)) for g in range(d)]
        mean_v = [plsc.load_gather(mean_b, [cvec[g] + 8]) for g in range(d)]
        std_v = [plsc.load_gather(std_b, [cvec[g] + 8]) for g in range(d)]
        is_len = [cvec[g] < 3 for g in range(d)]
        is_ang = [cvec[g] < 6 for g in range(d)]
        is_pos = [cvec[g] < 9 for g in range(d)]
        is_rot = [cvec[g] >= 9 for g in range(d)]

        def run_groups(buf, obuf):
          @plsc.parallel_loop(0, groups, unroll=2)
          def group(i):
            rows = i * 16 + iota
            a = [None] * d
            o = [None] * d
            q2 = jnp.full((16,), 1e-30, jnp.float32)
            for g in range(d):
                v = plsc.load_gather(buf, [rows, cvec[g]])
                a[g] = v * std_v[g] + mean_v[g]
                q2 = q2 + jnp.where(is_rot[g], a[g] * a[g], 0.0)
            norm = _sqrt16(q2) + 1e-8
            new_norm = jnp.minimum(jnp.maximum(norm, 0.01), _TWO_PI)
            f = new_norm / norm
            for g in range(d):
                lengths = jnp.maximum(jnp.abs(a[g]), 0.1)
                angles = jnp.minimum(jnp.maximum(a[g], 0.1), _PI - 0.1)
                pos = _frac(a[g])
                o[g] = jnp.where(
                    is_len[g],
                    lengths,
                    jnp.where(
                        is_ang[g],
                        angles,
                        jnp.where(is_pos[g], pos, a[g] * f),
                    ),
                )
                plsc.store_scatter(obuf, [rows, cvec[g]], o[g])

        def in_copy(c):
            c0 = r0 + c * chunk
            return pltpu.make_async_copy(
                noise_hbm.at[pl.ds(c0, chunk)], bufs[c % 2], isems[c % 2]
            )

        def out_copy(c):
            c0 = r0 + c * chunk
            return pltpu.make_async_copy(
                obufs[c % 2], out_hbm.at[pl.ds(c0, chunk)], osems[c % 2]
            )

        in_copy(0).start()
        if n_chunks > 1:
            in_copy(1).start()
        for c in range(n_chunks):
            in_copy(c).wait()
            if c >= 2:
                out_copy(c - 2).wait()
            run_groups(bufs[c % 2], obufs[c % 2])
            out_copy(c).start()
            if c + 2 < n_chunks:
                in_copy(c + 2).start()
        for c in range(max(0, n_chunks - 2), n_chunks):
            out_copy(c).wait()

    return sc_call


def kernel(num_samples, noise, sg_ind, means, stds):
    del sg_ind
    n, d = noise.shape
    num_workers = 32
    rows_w = n // num_workers
    sc_call = _make_sc_call(n, d, rows_w, chunk=128)
    return sc_call(noise, means, stds)


# speedup vs baseline: 1.0785x; 1.0785x over previous
import functools
import math

import jax
import jax.numpy as jnp
from jax import lax
from jax.experimental import pallas as pl
from jax.experimental.pallas import tpu as pltpu
from jax.experimental.pallas import tpu_sc as plsc

_PI = math.pi
_TWO_PI = 2.0 * math.pi


def _frac(x):
    r = lax.rem(x, jnp.float32(1.0))
    return jnp.where(r < 0.0, r + 1.0, r)


def _sqrt16(s):
    bits = plsc.bitcast(s, jnp.int32)
    seed = jnp.int32(0x5F3759DF) - lax.shift_right_logical(bits, 1)
    y = plsc.bitcast(seed, jnp.float32)
    for _ in range(3):
        y = y * (1.5 - 0.5 * s * y * y)
    return s * y


def _make_sc_call(n, d, rows_w, chunk):
    groups = chunk // 16
    n_chunks = rows_w // chunk
    mesh = plsc.VectorSubcoreMesh(core_axis_name="c", subcore_axis_name="s")

    @functools.partial(
        pl.kernel,
        out_type=jax.ShapeDtypeStruct((n, d), jnp.float32),
        mesh=mesh,
        scratch_types=[
            pltpu.VMEM((chunk, d), jnp.float32),
            pltpu.VMEM((chunk, d), jnp.float32),
            pltpu.VMEM((chunk, d), jnp.float32),
            pltpu.VMEM((chunk, d), jnp.float32),
            pltpu.SemaphoreType.DMA,
            pltpu.SemaphoreType.DMA,
            pltpu.SemaphoreType.DMA,
            pltpu.SemaphoreType.DMA,
            pltpu.VMEM((24,), jnp.float32),
            pltpu.VMEM((24,), jnp.float32),
        ],
        compiler_params=pltpu.CompilerParams(needs_layout_passes=False),
    )
    def sc_call(
        noise_hbm,
        means_hbm,
        stds_hbm,
        out_hbm,
        buf0,
        obuf0,
        buf1,
        obuf1,
        isem0,
        isem1,
        osem0,
        osem1,
        mean_b,
        std_b,
    ):
        bufs = (buf0, buf1)
        obufs = (obuf0, obuf1)
        isems = (isem0, isem1)
        osems = (osem0, osem1)
        nc = 2
        w = lax.axis_index("s") * nc + lax.axis_index("c")
        r0 = w * rows_w
        pltpu.sync_copy(means_hbm, mean_b.at[pl.ds(8, d)])
        pltpu.sync_copy(stds_hbm, std_b.at[pl.ds(8, d)])

        iota = lax.iota(jnp.int32, 16)
        cvec = [lax.rem(iota + g, jnp.int32(d)) for g in range(d)]
        mean_v = [plsc.load_gather(mean_b, [cvec[g] + 8]) for g in range(d)]
        std_v = [plsc.load_gather(std_b, [cvec[g] + 8]) for g in range(d)]
        is_len = [cvec[g] < 3 for g in range(d)]
        is_ang = [cvec[g] < 6 for g in range(d)]
        is_pos = [cvec[g] < 9 for g in range(d)]
        is_rot = [cvec[g] >= 9 for g in range(d)]

        def run_groups(buf, obuf):
          @plsc.parallel_loop(0, groups)
          def group(i):
            rows = i * 16 + iota
            a = [None] * d
            o = [None] * d
            q2 = jnp.full((16,), 1e-30, jnp.float32)
            for g in range(d):
                v = plsc.load_gather(buf, [rows, cvec[g]])
                a[g] = v * std_v[g] + mean_v[g]
                q2 = q2 + jnp.where(is_rot[g], a[g] * a[g], 0.0)
            norm = _sqrt16(q2) + 1e-8
            new_norm = jnp.minimum(jnp.maximum(norm, 0.01), _TWO_PI)
            f = new_norm / norm
            for g in range(d):
                lengths = jnp.maximum(jnp.abs(a[g]), 0.1)
                angles = jnp.minimum(jnp.maximum(a[g], 0.1), _PI - 0.1)
                pos = _frac(a[g])
                o[g] = jnp.where(
                    is_len[g],
                    lengths,
                    jnp.where(
                        is_ang[g],
                        angles,
                        jnp.where(is_pos[g], pos, a[g] * f),
                    ),
                )
                plsc.store_scatter(obuf, [rows, cvec[g]], o[g])

        def in_copy(c):
            c0 = r0 + c * chunk
            return pltpu.make_async_copy(
                noise_hbm.at[pl.ds(c0, chunk)], bufs[c % 2], isems[c % 2]
            )

        def out_copy(c):
            c0 = r0 + c * chunk
            return pltpu.make_async_copy(
                obufs[c % 2], out_hbm.at[pl.ds(c0, chunk)], osems[c % 2]
            )

        in_copy(0).start()
        if n_chunks > 1:
            in_copy(1).start()
        for c in range(n_chunks):
            in_copy(c).wait()
            if c >= 2:
                out_copy(c - 2).wait()
            run_groups(bufs[c % 2], obufs[c % 2])
            out_copy(c).start()
            if c + 2 < n_chunks:
                in_copy(c + 2).start()
        for c in range(max(0, n_chunks - 2), n_chunks):
            out_copy(c).wait()

    return sc_call


def kernel(num_samples, noise, sg_ind, means, stds):
    del sg_ind
    n, d = noise.shape
    num_workers = 32
    rows_w = n // num_workers
    sc_call = _make_sc_call(n, d, rows_w, chunk=128)
    return sc_call(noise, means, stds)


# speedup vs baseline: 1.1286x; 1.0465x over previous
import functools
import math

import jax
import jax.numpy as jnp
from jax import lax
from jax.experimental import pallas as pl
from jax.experimental.pallas import tpu as pltpu
from jax.experimental.pallas import tpu_sc as plsc

_PI = math.pi
_TWO_PI = 2.0 * math.pi


def _frac(x):
    r = lax.rem(x, jnp.float32(1.0))
    return jnp.where(r < 0.0, r + 1.0, r)


def _sqrt16(s):
    bits = plsc.bitcast(s, jnp.int32)
    seed = jnp.int32(0x5F3759DF) - lax.shift_right_logical(bits, 1)
    y = plsc.bitcast(seed, jnp.float32)
    for _ in range(3):
        y = y * (1.5 - 0.5 * s * y * y)
    return s * y


def _make_sc_call(n, d, rows_w, chunk):
    groups = chunk // 16
    n_chunks = rows_w // chunk
    mesh = plsc.VectorSubcoreMesh(core_axis_name="c", subcore_axis_name="s")

    @functools.partial(
        pl.kernel,
        out_type=jax.ShapeDtypeStruct((n, d), jnp.float32),
        mesh=mesh,
        scratch_types=[
            pltpu.VMEM((chunk, d), jnp.float32),
            pltpu.VMEM((chunk, d), jnp.float32),
            pltpu.VMEM((chunk, d), jnp.float32),
            pltpu.VMEM((chunk, d), jnp.float32),
            pltpu.SemaphoreType.DMA,
            pltpu.SemaphoreType.DMA,
            pltpu.SemaphoreType.DMA,
            pltpu.SemaphoreType.DMA,
            pltpu.VMEM((24,), jnp.float32),
            pltpu.VMEM((24,), jnp.float32),
        ],
        compiler_params=pltpu.CompilerParams(needs_layout_passes=False),
    )
    def sc_call(
        noise_hbm,
        means_hbm,
        stds_hbm,
        out_hbm,
        buf0,
        obuf0,
        buf1,
        obuf1,
        isem0,
        isem1,
        osem0,
        osem1,
        mean_b,
        std_b,
    ):
        bufs = (buf0, buf1)
        obufs = (obuf0, obuf1)
        isems = (isem0, isem1)
        osems = (osem0, osem1)
        nc = 2
        w = lax.axis_index("s") * nc + lax.axis_index("c")
        r0 = w * rows_w
        pltpu.sync_copy(means_hbm, mean_b.at[pl.ds(8, d)])
        pltpu.sync_copy(stds_hbm, std_b.at[pl.ds(8, d)])

        iota = lax.iota(jnp.int32, 16)
        cvec = [lax.rem(iota + g, jnp.int32(d)) for g in range(d)]
        mean_v = [plsc.load_gather(mean_b, [cvec[g] + 8]) for g in range(d)]
        std_v = [plsc.load_gather(std_b, [cvec[g] + 8]) for g in range(d)]
        is_len = [cvec[g] < 3 for g in range(d)]
        is_ang = [cvec[g] < 6 for g in range(d)]
        is_pos = [cvec[g] < 9 for g in range(d)]
        is_rot = [cvec[g] >= 9 for g in range(d)]

        def make_group(buf, obuf):
          def group(i, carry):
            rows = i * 16 + iota
            a = [None] * d
            o = [None] * d
            q2 = jnp.full((16,), 1e-30, jnp.float32)
            for g in range(d):
                v = plsc.load_gather(buf, [rows, cvec[g]])
                a[g] = v * std_v[g] + mean_v[g]
                q2 = q2 + jnp.where(is_rot[g], a[g] * a[g], 0.0)
            norm = _sqrt16(q2) + 1e-8
            new_norm = jnp.minimum(jnp.maximum(norm, 0.01), _TWO_PI)
            f = new_norm / norm
            for g in range(d):
                lengths = jnp.maximum(jnp.abs(a[g]), 0.1)
                angles = jnp.minimum(jnp.maximum(a[g], 0.1), _PI - 0.1)
                pos = _frac(a[g])
                o[g] = jnp.where(
                    is_len[g],
                    lengths,
                    jnp.where(
                        is_ang[g],
                        angles,
                        jnp.where(is_pos[g], pos, a[g] * f),
                    ),
                )
                plsc.store_scatter(obuf, [rows, cvec[g]], o[g])
            return carry

          return group

        group_fns = (make_group(bufs[0], obufs[0]), make_group(bufs[1], obufs[1]))

        def in_copy(c):
            c0 = r0 + c * chunk
            return pltpu.make_async_copy(
                noise_hbm.at[pl.ds(c0, chunk)], bufs[c % 2], isems[c % 2]
            )

        def out_copy(c):
            c0 = r0 + c * chunk
            return pltpu.make_async_copy(
                obufs[c % 2], out_hbm.at[pl.ds(c0, chunk)], osems[c % 2]
            )

        in_copy(0).start()
        if n_chunks > 1:
            in_copy(1).start()
        for c in range(n_chunks):
            in_copy(c).wait()
            if c >= 2:
                out_copy(c - 2).wait()
            lax.fori_loop(0, groups, group_fns[c % 2], 0)
            out_copy(c).start()
            if c + 2 < n_chunks:
                in_copy(c + 2).start()
        for c in range(max(0, n_chunks - 2), n_chunks):
            out_copy(c).wait()

    return sc_call


def kernel(num_samples, noise, sg_ind, means, stds):
    del sg_ind
    n, d = noise.shape
    num_workers = 32
    rows_w = n // num_workers
    sc_call = _make_sc_call(n, d, rows_w, chunk=128)
    return sc_call(noise, means, stds)
